# baseline (speedup 1.0000x reference)
import functools

import jax
import jax.numpy as jnp
from jax import lax
from jax.experimental import pallas as pl
from jax.experimental.pallas import tpu as pltpu
from jax.experimental.pallas import tpu_sc as plsc

D = 64
B = 8192
NC, NS, L = 2, 16, 16
NW = NC * NS
BPW = B // NW
CHUNK = 128
NCH = BPW // CHUNK

_mesh = plsc.VectorSubcoreMesh(core_axis_name="c", subcore_axis_name="s")


@functools.partial(
    pl.kernel,
    mesh=_mesh,
    out_type=jax.ShapeDtypeStruct((B, D), jnp.float32),
    scratch_types=[
        pltpu.VMEM((BPW,), jnp.int32),
        pltpu.VMEM((BPW, D), jnp.float32),
        pltpu.VMEM((BPW, D), jnp.float32),
        pltpu.SemaphoreType.DMA,
        pltpu.SemaphoreType.DMA,
    ],
    compiler_params=pltpu.CompilerParams(use_tc_tiling_on_sc=False),
)
def _emb_kernel(x_hbm, word_hbm, pos_hbm, out_hbm, idx_v, rows_v, pos_v,
                gsem, psem):
    wid = lax.axis_index("s") * NC + lax.axis_index("c")
    base = wid * BPW
    pltpu.sync_copy(x_hbm.at[pl.ds(base, BPW)], idx_v)
    pos_cp = pltpu.async_copy(pos_hbm.at[pl.ds(base, BPW)], pos_v, psem)
    gathers = []
    for j in range(NCH):
        sl = pl.ds(j * CHUNK, CHUNK)
        gathers.append(
            pltpu.async_copy(word_hbm.at[idx_v.at[sl]], rows_v.at[sl], gsem))
    pos_cp.wait()
    for cp in gathers:
        cp.wait()

    def body(r, _):
        for c in range(D // L):
            sl = pl.ds(c * L, L)
            rows_v[r, sl] = rows_v[r, sl] + pos_v[r, sl]
        return 0

    lax.fori_loop(0, BPW, body, 0)
    pltpu.sync_copy(rows_v, out_hbm.at[pl.ds(base, BPW)])


def kernel(x, word_table, pos_table):
    return _emb_kernel(x.astype(jnp.int32), word_table, pos_table[:B])


# speedup vs baseline: 1.6873x; 1.6873x over previous
import functools

import jax
import jax.numpy as jnp
from jax import lax
from jax.experimental import pallas as pl
from jax.experimental.pallas import tpu as pltpu
from jax.experimental.pallas import tpu_sc as plsc

D = 64
B = 8192
NC, NS, L = 2, 16, 16
NW = NC * NS
BPW = B // NW
K = 16
NCH = BPW // K

_mesh = plsc.VectorSubcoreMesh(core_axis_name="c", subcore_axis_name="s")


@functools.partial(
    pl.kernel,
    mesh=_mesh,
    out_type=jax.ShapeDtypeStruct((B, D), jnp.float32),
    scratch_types=[
        pltpu.VMEM((BPW,), jnp.int32),
        pltpu.VMEM((BPW, D), jnp.float32),
        pltpu.VMEM((BPW, D), jnp.float32),
        pltpu.SemaphoreType.DMA,
        pltpu.SemaphoreType.DMA,
        pltpu.SemaphoreType.DMA,
    ],
)
def _emb_kernel(x_hbm, word_hbm, pos_hbm, out_hbm, idx_v, rows_v, pos_v,
                sem_a, sem_b, psem):
    wid = lax.axis_index("s") * NC + lax.axis_index("c")
    base = wid * BPW
    pltpu.sync_copy(x_hbm.at[pl.ds(base, BPW)], idx_v)
    pos_cp = pltpu.async_copy(pos_hbm.at[pl.ds(base, BPW)], pos_v, psem)

    def issue(c, sem):
        iv = idx_v[pl.ds(c * K, K)]
        for j in range(K):
            r = c * K + j
            pltpu.async_copy(word_hbm.at[pl.ds(iv[j], 1)],
                             rows_v.at[pl.ds(r, 1)], sem)

    def drain(c, sem):
        pltpu.make_async_copy(word_hbm.at[pl.ds(0, K)],
                              rows_v.at[pl.ds(c * K, K)], sem).wait()

    sems = [sem_a, sem_b]
    issue(0, sems[0])
    for c in range(1, NCH):
        issue(c, sems[c % 2])
        drain(c - 1, sems[(c - 1) % 2])
    drain(NCH - 1, sems[(NCH - 1) % 2])
    pos_cp.wait()

    def body(r, _):
        for c in range(D // L):
            sl = pl.ds(c * L, L)
            rows_v[r, sl] = rows_v[r, sl] + pos_v[r, sl]
        return 0

    lax.fori_loop(0, BPW, body, 0)
    pltpu.sync_copy(rows_v, out_hbm.at[pl.ds(base, BPW)])


def kernel(x, word_table, pos_table):
    return _emb_kernel(x.astype(jnp.int32), word_table, pos_table[:B])


# speedup vs baseline: 1.6952x; 1.0046x over previous
import functools

import jax
import jax.numpy as jnp
from jax import lax
from jax.experimental import pallas as pl
from jax.experimental.pallas import tpu as pltpu
from jax.experimental.pallas import tpu_sc as plsc

D = 64
B = 8192
NC, NS, L = 2, 16, 16
NW = NC * NS
BPW = B // NW
K = 16
NCH = BPW // K
RING = 4

_mesh = plsc.VectorSubcoreMesh(core_axis_name="c", subcore_axis_name="s")


@functools.partial(
    pl.kernel,
    mesh=_mesh,
    out_type=jax.ShapeDtypeStruct((B, D), jnp.float32),
    scratch_types=[
        pltpu.VMEM((BPW,), jnp.int32),
        pltpu.VMEM((BPW, D), jnp.float32),
        pltpu.VMEM((BPW, D), jnp.float32),
        pltpu.SemaphoreType.DMA,
        pltpu.SemaphoreType.DMA,
        pltpu.SemaphoreType.DMA,
        pltpu.SemaphoreType.DMA,
        pltpu.SemaphoreType.DMA,
    ],
)
def _emb_kernel(x_hbm, word_hbm, pos_hbm, out_hbm, idx_v, rows_v, pos_v,
                sem0, sem1, sem2, sem3, psem):
    wid = lax.axis_index("s") * NC + lax.axis_index("c")
    base = wid * BPW
    pltpu.sync_copy(x_hbm.at[pl.ds(base, BPW)], idx_v)
    pos_cp = pltpu.async_copy(pos_hbm.at[pl.ds(base, BPW)], pos_v, psem)
    sems = [sem0, sem1, sem2, sem3]

    def issue(c):
        iv = idx_v[pl.ds(c * K, K)]
        for j in range(K):
            r = c * K + j
            pltpu.async_copy(word_hbm.at[pl.ds(iv[j], 1)],
                             rows_v.at[pl.ds(r, 1)], sems[c % RING])

    def drain(c):
        pltpu.make_async_copy(word_hbm.at[pl.ds(0, K)],
                              rows_v.at[pl.ds(c * K, K)],
                              sems[c % RING]).wait()

    for c in range(min(RING, NCH)):
        issue(c)
    for c in range(NCH):
        if c + RING < NCH:
            issue(c + RING)
        drain(c)
    pos_cp.wait()

    def body(r, _):
        for c in range(D // L):
            sl = pl.ds(c * L, L)
            rows_v[r, sl] = rows_v[r, sl] + pos_v[r, sl]
        return 0

    lax.fori_loop(0, BPW, body, 0)
    pltpu.sync_copy(rows_v, out_hbm.at[pl.ds(base, BPW)])


def kernel(x, word_table, pos_table):
    return _emb_kernel(x.astype(jnp.int32), word_table, pos_table[:B])


# speedup vs baseline: 2.0574x; 1.2137x over previous
import functools

import jax
import jax.numpy as jnp
from jax import lax
from jax.experimental import pallas as pl
from jax.experimental.pallas import tpu as pltpu
from jax.experimental.pallas import tpu_sc as plsc

V = 1000000
D = 64
B = 8192
NC, NS, L = 2, 16, 16
NW = NC * NS
BPW = B // NW
HALF = 1 << 19
RB = 4096
GRID = HALF // RB
NCB = (V + RB - 1) // RB - 1


def _transpose_pack(wt_T):
    def body(lo_ref, hi_ref, out_ref):
        out_ref[...] = jnp.concatenate(
            [lo_ref[...].T, hi_ref[...].T], axis=1)

    return pl.pallas_call(
        body,
        grid=(GRID,),
        in_specs=[
            pl.BlockSpec((D, RB), lambda b: (0, b)),
            pl.BlockSpec((D, RB), lambda b: (0, jnp.minimum(b + GRID, NCB))),
        ],
        out_specs=pl.BlockSpec((RB, 128), lambda b: (b, 0)),
        out_shape=jax.ShapeDtypeStruct((HALF, 128), jnp.float32),
    )(wt_T, wt_T)


_mesh = plsc.VectorSubcoreMesh(core_axis_name="c", subcore_axis_name="s")


@functools.partial(
    pl.kernel,
    mesh=_mesh,
    out_type=jax.ShapeDtypeStruct((B, D), jnp.float32),
    scratch_types=[
        pltpu.VMEM((BPW,), jnp.int32),
        pltpu.VMEM((BPW // L, L), jnp.int32),
        pltpu.VMEM((BPW, 128), jnp.float32),
        pltpu.VMEM((BPW, D), jnp.float32),
        pltpu.VMEM((BPW, D), jnp.float32),
        pltpu.SemaphoreType.DMA,
        pltpu.SemaphoreType.DMA,
    ],
)
def _sc_gather(pair_hbm, half_hbm, tab2_hbm, pos_hbm, out_hbm,
               pair_v, half_v, rows_v, pos_v, out_v, gsem, psem):
    wid = lax.axis_index("s") * NC + lax.axis_index("c")
    base = wid * BPW
    pltpu.sync_copy(pair_hbm.at[pl.ds(base, BPW)], pair_v)
    pltpu.sync_copy(half_hbm.at[pl.ds(wid * (BPW // L), BPW // L)], half_v)
    pos_cp = pltpu.async_copy(pos_hbm.at[pl.ds(base, BPW)], pos_v, psem)
    for j in range(BPW // 128):
        sl = pl.ds(j * 128, 128)
        pltpu.async_copy(tab2_hbm.at[pair_v.at[sl]], rows_v.at[sl], gsem)
    pos_cp.wait()
    for j in range(BPW // 128):
        sl = pl.ds(j * 128, 128)
        pltpu.make_async_copy(tab2_hbm.at[pair_v.at[sl]], rows_v.at[sl],
                              gsem).wait()

    def gbody(g, _):
        hv = half_v[g]
        for j in range(L):
            r = g * L + j
            h = hv[j]

            @pl.when(h == 0)
            def _lo():
                for q in range(D // L):
                    sl = pl.ds(q * L, L)
                    out_v[r, sl] = rows_v[r, sl] + pos_v[r, sl]

            @pl.when(h != 0)
            def _hi():
                for q in range(D // L):
                    sl = pl.ds(q * L, L)
                    out_v[r, sl] = rows_v[r, pl.ds(D + q * L, L)] + pos_v[r, sl]

        return 0

    lax.fori_loop(0, BPW // L, gbody, 0)
    pltpu.sync_copy(out_v, out_hbm.at[pl.ds(base, BPW)])


def kernel(x, word_table, pos_table):
    xi = x.astype(jnp.int32)
    tab2 = _transpose_pack(word_table.T)
    return _sc_gather(xi & (HALF - 1), (xi >> 19).reshape(B // L, L), tab2,
                      pos_table[:B])
